# baseline (speedup 1.0000x reference)
import jax
import jax.numpy as jnp
from jax import lax
from jax.experimental import pallas as pl
from jax.experimental.pallas import tpu as pltpu
from jax.experimental.pallas import tpu_sc as plsc

NC = 2
NS = 16
CHUNK = 128
NBUF = 4


def _zero_chunks(rows_per_tile):
    out = []
    off = 0
    while off < rows_per_tile:
        sz = min(128, rows_per_tile - off)
        out.append((off, sz))
        off += sz
    return out


def _sc_segsum(table, ei_flat, e, n_acc, pair_mode, with_count):
    m, dh = table.shape
    n = m // NC
    ept = e // NS
    cpt = ept // CHUNK
    tail = ept % CHUNK
    ring = cpt - cpt % NBUF
    rpt = n_acc // NS
    zchunks = _zero_chunks(rpt)

    mesh = plsc.VectorSubcoreMesh(core_axis_name="c", subcore_axis_name="s",
                                  num_cores=NC, num_subcores=NS)

    out_type = [jax.ShapeDtypeStruct((NC, n_acc, dh), jnp.float32)]
    scratch = [
        pltpu.VMEM((ept,), jnp.int32),
        pltpu.VMEM((ept,), jnp.int32),
        [pltpu.VMEM((CHUNK, dh), jnp.float32) for _ in range(NBUF)],
        pltpu.VMEM_SHARED((n_acc, dh), jnp.float32),
        [pltpu.SemaphoreType.DMA for _ in range(NBUF)],
        [pltpu.SemaphoreType.DMA for _ in range(NBUF)],
        pltpu.SemaphoreType.DMA,
        pltpu.SemaphoreType.DMA,
    ]
    if with_count:
        out_type.append(jax.ShapeDtypeStruct((n_acc, 16), jnp.float32))
        scratch += [
            pltpu.VMEM((CHUNK, 16), jnp.float32),
            pltpu.VMEM_SHARED((n_acc, 16), jnp.float32),
            [pltpu.SemaphoreType.DMA for _ in range(NBUF)],
        ]

    def body(table_h, ei_h, zdh_h, z16_h, o16_h, *refs):
        if with_count:
            (agg_o, cnt_o, src_v, dst_v, rows, acc, sg, ss, sidx, sz0,
             ov, cacc, scc) = refs
        else:
            agg_o, src_v, dst_v, rows, acc, sg, ss, sidx, sz0 = refs
        c = lax.axis_index("c")
        s = lax.axis_index("s")
        base = s * rpt

        d_src = pltpu.async_copy(ei_h.at[pl.ds(s * ept, ept)], src_v, sidx)
        d_dst = pltpu.async_copy(ei_h.at[pl.ds(e + s * ept, ept)], dst_v,
                                 sidx)
        zd = []
        for off, sz in zchunks:
            zd.append(pltpu.async_copy(
                zdh_h.at[pl.ds(0, sz)], acc.at[pl.ds(base + off, sz)], sz0))
        if with_count:
            d_ones = pltpu.async_copy(o16_h, ov, sidx)

            @pl.when(c == 0)
            def _():
                for off, sz in zchunks:
                    pltpu.async_copy(z16_h.at[pl.ds(0, sz)],
                                     cacc.at[pl.ds(base + off, sz)],
                                     sz0).wait()
            d_ones.wait()
        for d in zd:
            d.wait()
        d_src.wait()
        d_dst.wait()

        @pl.loop(0, ept // 16)
        def _(k):
            v = src_v[pl.ds(k * 16, 16)]
            if pair_mode:
                src_v[pl.ds(k * 16, 16)] = v * 2 + c
            else:
                src_v[pl.ds(k * 16, 16)] = v + c * n
        plsc.subcore_barrier()

        table = table_h

        def sidx_of(j):
            return src_v.at[pl.ds(j * CHUNK, CHUNK)]

        def didx_of(j):
            return dst_v.at[pl.ds(j * CHUNK, CHUNK)]

        def g_copy(j, b):
            return pltpu.make_async_copy(table.at[sidx_of(j)], rows[b], sg[b])

        def s_copy(j, b):
            return pltpu.make_async_copy(rows[b], acc.at[didx_of(j)], ss[b])

        def c_copy(j, b):
            return pltpu.make_async_copy(ov, cacc.at[didx_of(j)], scc[b])

        for b in range(NBUF):
            g_copy(b, b).start()

        @pl.loop(0, ring // NBUF)
        def _(i):
            for b in range(NBUF):
                j = NBUF * i + b
                bn = (b + 2) % NBUF
                g_copy(j, b).wait()
                s_copy(j, b).start(add=True)
                if with_count:
                    @pl.when(c == 0)
                    def _():
                        c_copy(j, b).start(add=True)

                @pl.when(jnp.logical_and(j >= 2, j + 2 < ring))
                def _():
                    s_copy(j - 2, bn).wait()
                    if with_count:
                        @pl.when(c == 0)
                        def _():
                            c_copy(j - 2, bn).wait()
                    g_copy(j + 2, bn).start()

        for b in range(NBUF):
            s_copy(ring - NBUF + b, b).wait()
        if with_count:
            @pl.when(c == 0)
            def _():
                for b in range(NBUF):
                    c_copy(ring - NBUF + b, b).wait()

        for j in range(ring, cpt):
            pltpu.sync_copy(table.at[sidx_of(j)], rows[0])
            pltpu.sync_copy(rows[0], acc.at[didx_of(j)], add=True)
            if with_count:
                @pl.when(c == 0)
                def _():
                    pltpu.sync_copy(ov, cacc.at[didx_of(j)], add=True)
        if tail:
            ts = src_v.at[pl.ds(cpt * CHUNK, tail)]
            td = dst_v.at[pl.ds(cpt * CHUNK, tail)]
            rt = rows[0].at[pl.ds(0, tail)]
            pltpu.sync_copy(table.at[ts], rt)
            pltpu.sync_copy(rt, acc.at[td], add=True)
            if with_count:
                @pl.when(c == 0)
                def _():
                    pltpu.sync_copy(ov.at[pl.ds(0, tail)], cacc.at[td],
                                    add=True)

        plsc.subcore_barrier()

        od = []
        for off, sz in zchunks:
            od.append(pltpu.async_copy(
                acc.at[pl.ds(base + off, sz)],
                agg_o.at[c].at[pl.ds(base + off, sz)], sz0))
        if with_count:
            @pl.when(c == 0)
            def _():
                for off, sz in zchunks:
                    pltpu.async_copy(cacc.at[pl.ds(base + off, sz)],
                                     cnt_o.at[pl.ds(base + off, sz)],
                                     sz0).wait()
        for d in od:
            d.wait()

    k = pl.kernel(body, out_type=tuple(out_type), mesh=mesh,
                  scratch_types=scratch,
                  compiler_params=pltpu.CompilerParams(
                      use_tc_tiling_on_sc=False))
    zdh = jnp.zeros((128, dh), jnp.float32)
    z16 = jnp.zeros((128, 16), jnp.float32)
    o16 = jnp.ones((CHUNK, 16), jnp.float32)
    return k(table, ei_flat, zdh, z16, o16)


def _tc_layer(aggv, inv_pk, hv, wlb, wrb, bpk, relu, nc2,
              unpack_out=False, block_rows=10000):
    h_raw = hv.ndim == 2
    n2 = hv.shape[0] // 2 if h_raw else hv.shape[1]
    rp = block_rows // 2

    def body(agg_ref, inv_ref, h_ref, wlb_ref, wrb_ref, b_ref, o_ref):
        a0 = agg_ref[0] * inv_ref[...]
        a1 = agg_ref[1] * inv_ref[...]
        if h_raw:
            xr = h_ref[...].reshape(rp, 2, h_ref.shape[-1])
            xe = xr[:, 0]
            xo = xr[:, 1]
        for c2 in range(nc2):
            acc = jnp.dot(a0, wlb_ref[c2, 0],
                          preferred_element_type=jnp.float32)
            acc = acc + jnp.dot(a1, wlb_ref[c2, 1],
                                preferred_element_type=jnp.float32)
            if h_raw:
                acc = acc + jnp.concatenate(
                    [jnp.dot(xe, wrb_ref[c2],
                             preferred_element_type=jnp.float32),
                     jnp.dot(xo, wrb_ref[c2],
                             preferred_element_type=jnp.float32)], axis=-1)
            else:
                acc = acc + jnp.dot(h_ref[0], wrb_ref[c2, 0],
                                    preferred_element_type=jnp.float32)
                acc = acc + jnp.dot(h_ref[1], wrb_ref[c2, 1],
                                    preferred_element_type=jnp.float32)
            acc = acc + b_ref[c2]
            if relu:
                acc = jnp.maximum(acc, 0.0)
            if unpack_out:
                ye = acc[:, :64]
                yo = acc[:, 64:]
                o_ref[...] = jnp.stack([ye, yo], axis=1).reshape(
                    block_rows, 64)
            else:
                o_ref[c2] = acc

    if h_raw:
        din = hv.shape[1]
        h_spec = pl.BlockSpec((block_rows, din), lambda i: (i, 0))
        wr_spec = pl.BlockSpec((nc2, din, 64), lambda i: (0, 0, 0))
    else:
        h_spec = pl.BlockSpec((NC, rp, 128), lambda i: (0, i, 0))
        wr_spec = pl.BlockSpec((nc2, NC, 128, 128), lambda i: (0, 0, 0, 0))

    return pl.pallas_call(
        body,
        grid=(n2 // rp,),
        in_specs=[
            pl.BlockSpec((NC, rp, 128), lambda i: (0, i, 0)),
            pl.BlockSpec((rp, 128), lambda i: (i, 0)),
            h_spec,
            pl.BlockSpec((nc2, NC, 128, 128), lambda i: (0, 0, 0, 0)),
            wr_spec,
            pl.BlockSpec((nc2, 1, 128), lambda i: (0, 0, 0)),
        ],
        out_specs=(pl.BlockSpec((block_rows, 64), lambda i: (i, 0))
                   if unpack_out else
                   pl.BlockSpec((nc2, rp, 128), lambda i: (0, i, 0))),
        out_shape=(jax.ShapeDtypeStruct((2 * n2, 64), jnp.float32)
                   if unpack_out else
                   jax.ShapeDtypeStruct((nc2, n2, 128), jnp.float32)),
    )(aggv, inv_pk, hv, wlb, wrb, bpk)


def _bd(m):
    z = jnp.zeros_like(m)
    return jnp.concatenate(
        [jnp.concatenate([m, z], axis=1), jnp.concatenate([z, m], axis=1)],
        axis=0)


def _pack_weights(wl, wr, b, nc2, h_raw=False):
    wlb = jnp.stack([
        jnp.stack([_bd(wl[c1 * 64:(c1 + 1) * 64, c2 * 64:(c2 + 1) * 64])
                   for c1 in range(NC)])
        for c2 in range(nc2)])
    if h_raw:
        wrb = jnp.stack([wr[:, c2 * 64:(c2 + 1) * 64] for c2 in range(nc2)])
    else:
        wrb = jnp.stack([
            jnp.stack([_bd(wr[c1 * 64:(c1 + 1) * 64, c2 * 64:(c2 + 1) * 64])
                       for c1 in range(NC)])
            for c2 in range(nc2)])
    bpk = jnp.stack([
        jnp.concatenate([b[c2 * 64:(c2 + 1) * 64]] * 2)[None, :]
        for c2 in range(nc2)])
    return wlb, wrb, bpk


def kernel(x, edge_index, Wl1, Wr1, b1, Wl2, Wr2, b2, Wl3, Wr3, b3):
    n, d = x.shape
    e = edge_index.shape[1]
    n_acc = ((n + 16) + NS * 8 - 1) // (NS * 8) * (NS * 8)

    src = edge_index[0]
    dst = edge_index[1]
    align = NS * 16
    if e % align:
        pad = align - e % align
        ar = jnp.arange(pad, dtype=jnp.int32)
        src = jnp.concatenate([src, (ar * 37) % n])
        dst = jnp.concatenate([dst, n + (ar % 16)])
        e += pad
    ei_flat = jnp.concatenate([src, dst]) if e != edge_index.shape[1] \
        else edge_index.reshape(2 * e)

    dh = d // NC
    agg1, cnt = _sc_segsum(x.reshape(NC * n, dh), ei_flat, e, n_acc,
                           pair_mode=True, with_count=True)
    inv = 1.0 / jnp.maximum(cnt[:n, 0], 1.0)
    inv_pk = jnp.broadcast_to(inv[:, None], (n, dh)).reshape(n // 2, 128)

    wb1 = _pack_weights(Wl1, Wr1, b1, NC, h_raw=True)
    wb2 = _pack_weights(Wl2, Wr2, b2, NC)
    wb3 = _pack_weights(Wl3, Wr3, b3, 1)

    def aggv(a):
        return a.reshape(NC, n_acc // 2, 128)

    h1 = _tc_layer(aggv(agg1), inv_pk, x, *wb1, relu=True, nc2=NC)
    (agg2,) = _sc_segsum(h1.reshape(NC * n, dh), ei_flat, e, n_acc,
                         pair_mode=False, with_count=False)
    h2 = _tc_layer(aggv(agg2), inv_pk, h1, *wb2, relu=True, nc2=NC)
    (agg3,) = _sc_segsum(h2.reshape(NC * n, dh), ei_flat, e, n_acc,
                         pair_mode=False, with_count=False)
    out = _tc_layer(aggv(agg3), inv_pk, h2, *wb3, relu=False, nc2=1,
                    unpack_out=True)
    return out


# speedup vs baseline: 1.0118x; 1.0118x over previous
import jax
import jax.numpy as jnp
from jax import lax
from jax.experimental import pallas as pl
from jax.experimental.pallas import tpu as pltpu
from jax.experimental.pallas import tpu_sc as plsc

NC = 2
NS = 16
CHUNK = 128
NBUF = 4


def _zero_chunks(rows_per_tile):
    out = []
    off = 0
    while off < rows_per_tile:
        sz = min(128, rows_per_tile - off)
        out.append((off, sz))
        off += sz
    return out


def _sc_segsum(table, ei_flat, e, n_acc, pair_mode, with_count):
    m, dh = table.shape
    n = m // NC
    ept = e // NS
    cpt = ept // CHUNK
    tail = ept % CHUNK
    ring = cpt - cpt % NBUF
    rpt = n_acc // NS
    zchunks = _zero_chunks(rpt)

    mesh = plsc.VectorSubcoreMesh(core_axis_name="c", subcore_axis_name="s",
                                  num_cores=NC, num_subcores=NS)

    out_type = [jax.ShapeDtypeStruct((NC, n_acc, dh), jnp.float32)]
    scratch = [
        pltpu.VMEM((ept,), jnp.int32),
        pltpu.VMEM((ept,), jnp.int32),
        [pltpu.VMEM((CHUNK, dh), jnp.float32) for _ in range(NBUF)],
        pltpu.VMEM_SHARED((n_acc, dh), jnp.float32),
        [pltpu.SemaphoreType.DMA for _ in range(NBUF)],
        [pltpu.SemaphoreType.DMA for _ in range(NBUF)],
        pltpu.SemaphoreType.DMA,
        pltpu.SemaphoreType.DMA,
    ]
    if with_count:
        out_type.append(jax.ShapeDtypeStruct((n_acc, 16), jnp.float32))
        scratch += [
            pltpu.VMEM((CHUNK, 16), jnp.float32),
            pltpu.VMEM_SHARED((n_acc, 16), jnp.float32),
            [pltpu.SemaphoreType.DMA for _ in range(NBUF)],
        ]

    def body(table_h, ei_h, zdh_h, z16_h, o16_h, *refs):
        if with_count:
            (agg_o, cnt_o, src_v, dst_v, rows, acc, sg, ss, sidx, sz0,
             ov, cacc, scc) = refs
        else:
            agg_o, src_v, dst_v, rows, acc, sg, ss, sidx, sz0 = refs
        c = lax.axis_index("c")
        s = lax.axis_index("s")
        base = s * rpt

        d_src = pltpu.async_copy(ei_h.at[pl.ds(s * ept, ept)], src_v, sidx)
        d_dst = pltpu.async_copy(ei_h.at[pl.ds(e + s * ept, ept)], dst_v,
                                 sidx)
        zd = []
        for off, sz in zchunks:
            zd.append(pltpu.async_copy(
                zdh_h.at[pl.ds(0, sz)], acc.at[pl.ds(base + off, sz)], sz0))
        if with_count:
            d_ones = pltpu.async_copy(o16_h, ov, sidx)

            @pl.when(c == 0)
            def _():
                for off, sz in zchunks:
                    pltpu.async_copy(z16_h.at[pl.ds(0, sz)],
                                     cacc.at[pl.ds(base + off, sz)],
                                     sz0).wait()
            d_ones.wait()
        for d in zd:
            d.wait()
        d_src.wait()
        d_dst.wait()

        @pl.loop(0, ept // 16)
        def _(k):
            v = src_v[pl.ds(k * 16, 16)]
            if pair_mode:
                src_v[pl.ds(k * 16, 16)] = v * 2 + c
            else:
                src_v[pl.ds(k * 16, 16)] = v + c * n
        plsc.subcore_barrier()

        table = table_h

        def sidx_of(j):
            return src_v.at[pl.ds(j * CHUNK, CHUNK)]

        def didx_of(j):
            return dst_v.at[pl.ds(j * CHUNK, CHUNK)]

        def g_copy(j, b):
            return pltpu.make_async_copy(table.at[sidx_of(j)], rows[b], sg[b])

        def s_copy(j, b):
            return pltpu.make_async_copy(rows[b], acc.at[didx_of(j)], ss[b])

        def c_copy(j, b):
            return pltpu.make_async_copy(ov, cacc.at[didx_of(j)], scc[b])

        for b in range(NBUF):
            g_copy(b, b).start()

        @pl.loop(0, ring // NBUF)
        def _(i):
            for b in range(NBUF):
                j = NBUF * i + b
                bn = (b + 2) % NBUF
                g_copy(j, b).wait()
                s_copy(j, b).start(add=True)
                if with_count:
                    @pl.when(c == 0)
                    def _():
                        c_copy(j, b).start(add=True)

                @pl.when(jnp.logical_and(j >= 2, j + 2 < ring))
                def _():
                    s_copy(j - 2, bn).wait()
                    if with_count:
                        @pl.when(c == 0)
                        def _():
                            c_copy(j - 2, bn).wait()
                    g_copy(j + 2, bn).start()

        for b in range(NBUF):
            s_copy(ring - NBUF + b, b).wait()
        if with_count:
            @pl.when(c == 0)
            def _():
                for b in range(NBUF):
                    c_copy(ring - NBUF + b, b).wait()

        for j in range(ring, cpt):
            pltpu.sync_copy(table.at[sidx_of(j)], rows[0])
            pltpu.sync_copy(rows[0], acc.at[didx_of(j)], add=True)
            if with_count:
                @pl.when(c == 0)
                def _():
                    pltpu.sync_copy(ov, cacc.at[didx_of(j)], add=True)
        if tail:
            ts = src_v.at[pl.ds(cpt * CHUNK, tail)]
            td = dst_v.at[pl.ds(cpt * CHUNK, tail)]
            rt = rows[0].at[pl.ds(0, tail)]
            pltpu.sync_copy(table.at[ts], rt)
            pltpu.sync_copy(rt, acc.at[td], add=True)
            if with_count:
                @pl.when(c == 0)
                def _():
                    pltpu.sync_copy(ov.at[pl.ds(0, tail)], cacc.at[td],
                                    add=True)

        plsc.subcore_barrier()

        od = []
        for off, sz in zchunks:
            od.append(pltpu.async_copy(
                acc.at[pl.ds(base + off, sz)],
                agg_o.at[c].at[pl.ds(base + off, sz)], sz0))
        if with_count:
            @pl.when(c == 0)
            def _():
                for off, sz in zchunks:
                    pltpu.async_copy(cacc.at[pl.ds(base + off, sz)],
                                     cnt_o.at[pl.ds(base + off, sz)],
                                     sz0).wait()
        for d in od:
            d.wait()

    k = pl.kernel(body, out_type=tuple(out_type), mesh=mesh,
                  scratch_types=scratch,
                  compiler_params=pltpu.CompilerParams(
                      use_tc_tiling_on_sc=False))
    zdh = jnp.zeros((128, dh), jnp.float32)
    z16 = jnp.zeros((128, 16), jnp.float32)
    o16 = jnp.ones((CHUNK, 16), jnp.float32)
    return k(table, ei_flat, zdh, z16, o16)


def _tc_layer(aggv, inv_pk, hv, wlb, wrb, bpk, relu, nc2,
              unpack_out=False, block_rows=2000):
    h_raw = hv.ndim == 2
    n2 = hv.shape[0] // 2 if h_raw else hv.shape[1]
    rp = block_rows // 2

    def body(agg_ref, inv_ref, h_ref, wlb_ref, wrb_ref, b_ref, o_ref):
        a0 = agg_ref[0] * inv_ref[...]
        a1 = agg_ref[1] * inv_ref[...]
        if h_raw:
            xr = h_ref[...].reshape(rp, 2, h_ref.shape[-1])
            xe = xr[:, 0]
            xo = xr[:, 1]
        for c2 in range(nc2):
            acc = jnp.dot(a0, wlb_ref[c2, 0],
                          preferred_element_type=jnp.float32)
            acc = acc + jnp.dot(a1, wlb_ref[c2, 1],
                                preferred_element_type=jnp.float32)
            if h_raw:
                acc = acc + jnp.concatenate(
                    [jnp.dot(xe, wrb_ref[c2],
                             preferred_element_type=jnp.float32),
                     jnp.dot(xo, wrb_ref[c2],
                             preferred_element_type=jnp.float32)], axis=-1)
            else:
                acc = acc + jnp.dot(h_ref[0], wrb_ref[c2, 0],
                                    preferred_element_type=jnp.float32)
                acc = acc + jnp.dot(h_ref[1], wrb_ref[c2, 1],
                                    preferred_element_type=jnp.float32)
            acc = acc + b_ref[c2]
            if relu:
                acc = jnp.maximum(acc, 0.0)
            if unpack_out:
                ye = acc[:, :64]
                yo = acc[:, 64:]
                o_ref[...] = jnp.stack([ye, yo], axis=1).reshape(
                    block_rows, 64)
            else:
                o_ref[c2] = acc

    if h_raw:
        din = hv.shape[1]
        h_spec = pl.BlockSpec((block_rows, din), lambda i: (i, 0))
        wr_spec = pl.BlockSpec((nc2, din, 64), lambda i: (0, 0, 0))
    else:
        h_spec = pl.BlockSpec((NC, rp, 128), lambda i: (0, i, 0))
        wr_spec = pl.BlockSpec((nc2, NC, 128, 128), lambda i: (0, 0, 0, 0))

    return pl.pallas_call(
        body,
        grid=(n2 // rp,),
        in_specs=[
            pl.BlockSpec((NC, rp, 128), lambda i: (0, i, 0)),
            pl.BlockSpec((rp, 128), lambda i: (i, 0)),
            h_spec,
            pl.BlockSpec((nc2, NC, 128, 128), lambda i: (0, 0, 0, 0)),
            wr_spec,
            pl.BlockSpec((nc2, 1, 128), lambda i: (0, 0, 0)),
        ],
        out_specs=(pl.BlockSpec((block_rows, 64), lambda i: (i, 0))
                   if unpack_out else
                   pl.BlockSpec((nc2, rp, 128), lambda i: (0, i, 0))),
        out_shape=(jax.ShapeDtypeStruct((2 * n2, 64), jnp.float32)
                   if unpack_out else
                   jax.ShapeDtypeStruct((nc2, n2, 128), jnp.float32)),
    )(aggv, inv_pk, hv, wlb, wrb, bpk)


def _bd(m):
    z = jnp.zeros_like(m)
    return jnp.concatenate(
        [jnp.concatenate([m, z], axis=1), jnp.concatenate([z, m], axis=1)],
        axis=0)


def _pack_weights(wl, wr, b, nc2, h_raw=False):
    wlb = jnp.stack([
        jnp.stack([_bd(wl[c1 * 64:(c1 + 1) * 64, c2 * 64:(c2 + 1) * 64])
                   for c1 in range(NC)])
        for c2 in range(nc2)])
    if h_raw:
        wrb = jnp.stack([wr[:, c2 * 64:(c2 + 1) * 64] for c2 in range(nc2)])
    else:
        wrb = jnp.stack([
            jnp.stack([_bd(wr[c1 * 64:(c1 + 1) * 64, c2 * 64:(c2 + 1) * 64])
                       for c1 in range(NC)])
            for c2 in range(nc2)])
    bpk = jnp.stack([
        jnp.concatenate([b[c2 * 64:(c2 + 1) * 64]] * 2)[None, :]
        for c2 in range(nc2)])
    return wlb, wrb, bpk


def kernel(x, edge_index, Wl1, Wr1, b1, Wl2, Wr2, b2, Wl3, Wr3, b3):
    n, d = x.shape
    e = edge_index.shape[1]
    n_acc = ((n + 16) + NS * 8 - 1) // (NS * 8) * (NS * 8)

    src = edge_index[0]
    dst = edge_index[1]
    align = NS * 16
    if e % align:
        pad = align - e % align
        ar = jnp.arange(pad, dtype=jnp.int32)
        src = jnp.concatenate([src, (ar * 37) % n])
        dst = jnp.concatenate([dst, n + (ar % 16)])
        e += pad
    ei_flat = jnp.concatenate([src, dst]) if e != edge_index.shape[1] \
        else edge_index.reshape(2 * e)

    dh = d // NC
    agg1, cnt = _sc_segsum(x.reshape(NC * n, dh), ei_flat, e, n_acc,
                           pair_mode=True, with_count=True)
    inv = 1.0 / jnp.maximum(cnt[:n, 0], 1.0)
    inv_pk = jnp.broadcast_to(inv[:, None], (n, dh)).reshape(n // 2, 128)

    wb1 = _pack_weights(Wl1, Wr1, b1, NC, h_raw=True)
    wb2 = _pack_weights(Wl2, Wr2, b2, NC)
    wb3 = _pack_weights(Wl3, Wr3, b3, 1)

    def aggv(a):
        return a.reshape(NC, n_acc // 2, 128)

    h1 = _tc_layer(aggv(agg1), inv_pk, x, *wb1, relu=True, nc2=NC)
    (agg2,) = _sc_segsum(h1.reshape(NC * n, dh), ei_flat, e, n_acc,
                         pair_mode=False, with_count=False)
    h2 = _tc_layer(aggv(agg2), inv_pk, h1, *wb2, relu=True, nc2=NC)
    (agg3,) = _sc_segsum(h2.reshape(NC * n, dh), ei_flat, e, n_acc,
                         pair_mode=False, with_count=False)
    out = _tc_layer(aggv(agg3), inv_pk, h2, *wb3, relu=False, nc2=1,
                    unpack_out=True)
    return out


# speedup vs baseline: 1.0552x; 1.0429x over previous
import jax
import jax.numpy as jnp
from jax import lax
from jax.experimental import pallas as pl
from jax.experimental.pallas import tpu as pltpu
from jax.experimental.pallas import tpu_sc as plsc

NC = 2
NS = 16
CHUNK = 128
NBUF = 4
HALF = NBUF // 2


def _zero_chunks(rows_per_tile):
    out = []
    off = 0
    while off < rows_per_tile:
        sz = min(128, rows_per_tile - off)
        out.append((off, sz))
        off += sz
    return out


def _sc_segsum(table, ei_flat, e, n_acc, pair_mode, with_count):
    m, dh = table.shape
    n = m // NC
    ept = e // NS
    cpt = ept // CHUNK
    tail = ept % CHUNK
    ring = cpt - cpt % NBUF
    rpt = n_acc // NS
    zchunks = _zero_chunks(rpt)

    mesh = plsc.VectorSubcoreMesh(core_axis_name="c", subcore_axis_name="s",
                                  num_cores=NC, num_subcores=NS)

    out_type = [jax.ShapeDtypeStruct((NC, n_acc, dh), jnp.float32)]
    scratch = [
        pltpu.VMEM((ept,), jnp.int32),
        pltpu.VMEM((ept,), jnp.int32),
        [pltpu.VMEM((CHUNK, dh), jnp.float32) for _ in range(NBUF)],
        pltpu.VMEM_SHARED((n_acc, dh), jnp.float32),
        [pltpu.SemaphoreType.DMA for _ in range(NBUF)],
        [pltpu.SemaphoreType.DMA for _ in range(NBUF)],
        pltpu.SemaphoreType.DMA,
        pltpu.SemaphoreType.DMA,
    ]
    if with_count:
        out_type.append(jax.ShapeDtypeStruct((n_acc, 16), jnp.float32))
        scratch += [
            pltpu.VMEM((CHUNK, 16), jnp.float32),
            pltpu.VMEM_SHARED((n_acc, 16), jnp.float32),
            [pltpu.SemaphoreType.DMA for _ in range(NBUF)],
        ]

    def body(table_h, ei_h, zdh_h, z16_h, o16_h, *refs):
        if with_count:
            (agg_o, cnt_o, src_v, dst_v, rows, acc, sg, ss, sidx, sz0,
             ov, cacc, scc) = refs
        else:
            agg_o, src_v, dst_v, rows, acc, sg, ss, sidx, sz0 = refs
        c = lax.axis_index("c")
        s = lax.axis_index("s")
        base = s * rpt

        d_src = pltpu.async_copy(ei_h.at[pl.ds(s * ept, ept)], src_v, sidx)
        d_dst = pltpu.async_copy(ei_h.at[pl.ds(e + s * ept, ept)], dst_v,
                                 sidx)
        zd = []
        for off, sz in zchunks:
            zd.append(pltpu.async_copy(
                zdh_h.at[pl.ds(0, sz)], acc.at[pl.ds(base + off, sz)], sz0))
        if with_count:
            d_ones = pltpu.async_copy(o16_h, ov, sidx)

            @pl.when(c == 0)
            def _():
                for off, sz in zchunks:
                    pltpu.async_copy(z16_h.at[pl.ds(0, sz)],
                                     cacc.at[pl.ds(base + off, sz)], sz0)
            d_ones.wait()
        d_src.wait()
        d_dst.wait()

        def rebase(lo, nelem):
            for kk in range(nelem // 16):
                sl = pl.ds(lo + kk * 16, 16)
                v = src_v[sl]
                src_v[sl] = v * 2 + c if pair_mode else v + c * n

        table = table_h

        def sidx_of(j):
            return src_v.at[pl.ds(j * CHUNK, CHUNK)]

        def didx_of(j):
            return dst_v.at[pl.ds(j * CHUNK, CHUNK)]

        def g_copy(j, b):
            return pltpu.make_async_copy(table.at[sidx_of(j)], rows[b], sg[b])

        def s_copy(j, b):
            return pltpu.make_async_copy(rows[b], acc.at[didx_of(j)], ss[b])

        def c_copy(j, b):
            return pltpu.make_async_copy(ov, cacc.at[didx_of(j)], scc[b])

        for b in range(NBUF):
            rebase(b * CHUNK, CHUNK)
            g_copy(b, b).start()
        for j in range(ring, cpt):
            rebase(j * CHUNK, CHUNK)
        if tail:
            rebase(cpt * CHUNK, tail)

        for d in zd:
            d.wait()
        if with_count:
            @pl.when(c == 0)
            def _():
                for off, sz in zchunks:
                    pltpu.make_async_copy(
                        z16_h.at[pl.ds(0, sz)],
                        cacc.at[pl.ds(base + off, sz)], sz0).wait()
        plsc.subcore_barrier()

        @pl.loop(0, ring // NBUF)
        def _(i):
            for b in range(NBUF):
                j = NBUF * i + b
                bn = (b + HALF) % NBUF
                g_copy(j, b).wait()
                s_copy(j, b).start(add=True)
                if with_count:
                    @pl.when(c == 0)
                    def _():
                        c_copy(j, b).start(add=True)

                @pl.when(jnp.logical_and(j >= HALF, j + HALF < ring))
                def _():
                    s_copy(j - HALF, bn).wait()
                    if with_count:
                        @pl.when(c == 0)
                        def _():
                            c_copy(j - HALF, bn).wait()
                    rebase((j + HALF) * CHUNK, CHUNK)
                    g_copy(j + HALF, bn).start()

        for b in range(NBUF):
            s_copy(ring - NBUF + b, b).wait()
        if with_count:
            @pl.when(c == 0)
            def _():
                for b in range(NBUF):
                    c_copy(ring - NBUF + b, b).wait()

        for j in range(ring, cpt):
            pltpu.sync_copy(table.at[sidx_of(j)], rows[0])
            pltpu.sync_copy(rows[0], acc.at[didx_of(j)], add=True)
            if with_count:
                @pl.when(c == 0)
                def _():
                    pltpu.sync_copy(ov, cacc.at[didx_of(j)], add=True)
        if tail:
            ts = src_v.at[pl.ds(cpt * CHUNK, tail)]
            td = dst_v.at[pl.ds(cpt * CHUNK, tail)]
            rt = rows[0].at[pl.ds(0, tail)]
            pltpu.sync_copy(table.at[ts], rt)
            pltpu.sync_copy(rt, acc.at[td], add=True)
            if with_count:
                @pl.when(c == 0)
                def _():
                    pltpu.sync_copy(ov.at[pl.ds(0, tail)], cacc.at[td],
                                    add=True)

        plsc.subcore_barrier()

        od = []
        for off, sz in zchunks:
            od.append(pltpu.async_copy(
                acc.at[pl.ds(base + off, sz)],
                agg_o.at[c].at[pl.ds(base + off, sz)], sz0))
        if with_count:
            @pl.when(c == 0)
            def _():
                for off, sz in zchunks:
                    pltpu.async_copy(cacc.at[pl.ds(base + off, sz)],
                                     cnt_o.at[pl.ds(base + off, sz)],
                                     sz0).wait()
        for d in od:
            d.wait()

    k = pl.kernel(body, out_type=tuple(out_type), mesh=mesh,
                  scratch_types=scratch,
                  compiler_params=pltpu.CompilerParams(
                      use_tc_tiling_on_sc=False))
    zdh = jnp.zeros((128, dh), jnp.float32)
    z16 = jnp.zeros((128, 16), jnp.float32)
    o16 = jnp.ones((CHUNK, 16), jnp.float32)
    return k(table, ei_flat, zdh, z16, o16)


def _tc_layer(aggv, inv_pk, hv, wlb, wrb, bpk, relu, nc2,
              unpack_out=False, block_rows=2000):
    h_raw = hv.ndim == 2
    n2 = hv.shape[0] // 2 if h_raw else hv.shape[1]
    rp = block_rows // 2

    def body(agg_ref, inv_ref, h_ref, wlb_ref, wrb_ref, b_ref, o_ref):
        a0 = agg_ref[0] * inv_ref[...]
        a1 = agg_ref[1] * inv_ref[...]
        if h_raw:
            xr = h_ref[...].reshape(rp, 2, h_ref.shape[-1])
            xe = xr[:, 0]
            xo = xr[:, 1]
        for c2 in range(nc2):
            acc = jnp.dot(a0, wlb_ref[c2, 0],
                          preferred_element_type=jnp.float32)
            acc = acc + jnp.dot(a1, wlb_ref[c2, 1],
                                preferred_element_type=jnp.float32)
            if h_raw:
                acc = acc + jnp.concatenate(
                    [jnp.dot(xe, wrb_ref[c2],
                             preferred_element_type=jnp.float32),
                     jnp.dot(xo, wrb_ref[c2],
                             preferred_element_type=jnp.float32)], axis=-1)
            else:
                acc = acc + jnp.dot(h_ref[0], wrb_ref[c2, 0],
                                    preferred_element_type=jnp.float32)
                acc = acc + jnp.dot(h_ref[1], wrb_ref[c2, 1],
                                    preferred_element_type=jnp.float32)
            acc = acc + b_ref[c2]
            if relu:
                acc = jnp.maximum(acc, 0.0)
            if unpack_out:
                ye = acc[:, :64]
                yo = acc[:, 64:]
                o_ref[...] = jnp.stack([ye, yo], axis=1).reshape(
                    block_rows, 64)
            else:
                o_ref[c2] = acc

    if h_raw:
        din = hv.shape[1]
        h_spec = pl.BlockSpec((block_rows, din), lambda i: (i, 0))
        wr_spec = pl.BlockSpec((nc2, din, 64), lambda i: (0, 0, 0))
    else:
        h_spec = pl.BlockSpec((NC, rp, 128), lambda i: (0, i, 0))
        wr_spec = pl.BlockSpec((nc2, NC, 128, 128), lambda i: (0, 0, 0, 0))

    return pl.pallas_call(
        body,
        grid=(n2 // rp,),
        in_specs=[
            pl.BlockSpec((NC, rp, 128), lambda i: (0, i, 0)),
            pl.BlockSpec((rp, 128), lambda i: (i, 0)),
            h_spec,
            pl.BlockSpec((nc2, NC, 128, 128), lambda i: (0, 0, 0, 0)),
            wr_spec,
            pl.BlockSpec((nc2, 1, 128), lambda i: (0, 0, 0)),
        ],
        out_specs=(pl.BlockSpec((block_rows, 64), lambda i: (i, 0))
                   if unpack_out else
                   pl.BlockSpec((nc2, rp, 128), lambda i: (0, i, 0))),
        out_shape=(jax.ShapeDtypeStruct((2 * n2, 64), jnp.float32)
                   if unpack_out else
                   jax.ShapeDtypeStruct((nc2, n2, 128), jnp.float32)),
    )(aggv, inv_pk, hv, wlb, wrb, bpk)


def _bd(m):
    z = jnp.zeros_like(m)
    return jnp.concatenate(
        [jnp.concatenate([m, z], axis=1), jnp.concatenate([z, m], axis=1)],
        axis=0)


def _pack_weights(wl, wr, b, nc2, h_raw=False):
    wlb = jnp.stack([
        jnp.stack([_bd(wl[c1 * 64:(c1 + 1) * 64, c2 * 64:(c2 + 1) * 64])
                   for c1 in range(NC)])
        for c2 in range(nc2)])
    if h_raw:
        wrb = jnp.stack([wr[:, c2 * 64:(c2 + 1) * 64] for c2 in range(nc2)])
    else:
        wrb = jnp.stack([
            jnp.stack([_bd(wr[c1 * 64:(c1 + 1) * 64, c2 * 64:(c2 + 1) * 64])
                       for c1 in range(NC)])
            for c2 in range(nc2)])
    bpk = jnp.stack([
        jnp.concatenate([b[c2 * 64:(c2 + 1) * 64]] * 2)[None, :]
        for c2 in range(nc2)])
    return wlb, wrb, bpk


def kernel(x, edge_index, Wl1, Wr1, b1, Wl2, Wr2, b2, Wl3, Wr3, b3):
    n, d = x.shape
    e = edge_index.shape[1]
    n_acc = ((n + 16) + NS * 8 - 1) // (NS * 8) * (NS * 8)

    src = edge_index[0]
    dst = edge_index[1]
    align = NS * 16
    if e % align:
        pad = align - e % align
        ar = jnp.arange(pad, dtype=jnp.int32)
        src = jnp.concatenate([src, (ar * 37) % n])
        dst = jnp.concatenate([dst, n + (ar % 16)])
        e += pad
    ei_flat = jnp.concatenate([src, dst]) if e != edge_index.shape[1] \
        else edge_index.reshape(2 * e)

    dh = d // NC
    agg1, cnt = _sc_segsum(x.reshape(NC * n, dh), ei_flat, e, n_acc,
                           pair_mode=True, with_count=True)
    inv = 1.0 / jnp.maximum(cnt[:n, 0], 1.0)
    inv_pk = jnp.broadcast_to(inv[:, None], (n, dh)).reshape(n // 2, 128)

    wb1 = _pack_weights(Wl1, Wr1, b1, NC, h_raw=True)
    wb2 = _pack_weights(Wl2, Wr2, b2, NC)
    wb3 = _pack_weights(Wl3, Wr3, b3, 1)

    def aggv(a):
        return a.reshape(NC, n_acc // 2, 128)

    h1 = _tc_layer(aggv(agg1), inv_pk, x, *wb1, relu=True, nc2=NC)
    (agg2,) = _sc_segsum(h1.reshape(NC * n, dh), ei_flat, e, n_acc,
                         pair_mode=False, with_count=False)
    h2 = _tc_layer(aggv(agg2), inv_pk, h1, *wb2, relu=True, nc2=NC)
    (agg3,) = _sc_segsum(h2.reshape(NC * n, dh), ei_flat, e, n_acc,
                         pair_mode=False, with_count=False)
    out = _tc_layer(aggv(agg3), inv_pk, h2, *wb3, relu=False, nc2=1)
    return out.reshape(n, dh)


# speedup vs baseline: 1.1091x; 1.0511x over previous
import jax
import jax.numpy as jnp
from jax import lax
from jax.experimental import pallas as pl
from jax.experimental.pallas import tpu as pltpu
from jax.experimental.pallas import tpu_sc as plsc

NC = 2
NS = 16
CHUNK = 128
NBUF = 4
HALF = NBUF // 2


def _zero_chunks(rows_per_tile):
    out = []
    off = 0
    while off < rows_per_tile:
        sz = min(128, rows_per_tile - off)
        out.append((off, sz))
        off += sz
    return out


def _sc_segsum(table, ei_flat, e, n_acc, pair_mode, with_count,
               NBUF=NBUF):
    HALF = NBUF // 2
    m, dh = table.shape
    n = m // NC
    ept = e // NS
    cpt = ept // CHUNK
    tail = ept % CHUNK
    ring = cpt - cpt % NBUF
    rpt = n_acc // NS
    zchunks = _zero_chunks(rpt)

    mesh = plsc.VectorSubcoreMesh(core_axis_name="c", subcore_axis_name="s",
                                  num_cores=NC, num_subcores=NS)

    out_type = [jax.ShapeDtypeStruct((NC, n_acc, dh), jnp.float32)]
    scratch = [
        pltpu.VMEM((ept,), jnp.int32),
        pltpu.VMEM((ept,), jnp.int32),
        [pltpu.VMEM((CHUNK, dh), jnp.float32) for _ in range(NBUF)],
        pltpu.VMEM_SHARED((n_acc, dh), jnp.float32),
        [pltpu.SemaphoreType.DMA for _ in range(NBUF)],
        [pltpu.SemaphoreType.DMA for _ in range(NBUF)],
        pltpu.SemaphoreType.DMA,
        pltpu.SemaphoreType.DMA,
    ]
    if with_count:
        out_type.append(jax.ShapeDtypeStruct((n_acc, 16), jnp.float32))
        scratch += [
            pltpu.VMEM((CHUNK, 16), jnp.float32),
            pltpu.VMEM_SHARED((n_acc, 16), jnp.float32),
            [pltpu.SemaphoreType.DMA for _ in range(NBUF)],
        ]

    def body(table_h, ei_h, zdh_h, z16_h, o16_h, *refs):
        if with_count:
            (agg_o, cnt_o, src_v, dst_v, rows, acc, sg, ss, sidx, sz0,
             ov, cacc, scc) = refs
        else:
            agg_o, src_v, dst_v, rows, acc, sg, ss, sidx, sz0 = refs
        c = lax.axis_index("c")
        s = lax.axis_index("s")
        base = s * rpt

        d_src = pltpu.async_copy(ei_h.at[pl.ds(s * ept, ept)], src_v, sidx)
        d_dst = pltpu.async_copy(ei_h.at[pl.ds(e + s * ept, ept)], dst_v,
                                 sidx)
        zd = []
        for off, sz in zchunks:
            zd.append(pltpu.async_copy(
                zdh_h.at[pl.ds(0, sz)], acc.at[pl.ds(base + off, sz)], sz0))
        if with_count:
            d_ones = pltpu.async_copy(o16_h, ov, sidx)

            @pl.when(c == 0)
            def _():
                for off, sz in zchunks:
                    pltpu.async_copy(z16_h.at[pl.ds(0, sz)],
                                     cacc.at[pl.ds(base + off, sz)], sz0)
            d_ones.wait()
        d_src.wait()
        d_dst.wait()

        def rebase(lo, nelem):
            for kk in range(nelem // 16):
                sl = pl.ds(lo + kk * 16, 16)
                v = src_v[sl]
                src_v[sl] = v * 2 + c if pair_mode else v + c * n

        table = table_h

        def sidx_of(j):
            return src_v.at[pl.ds(j * CHUNK, CHUNK)]

        def didx_of(j):
            return dst_v.at[pl.ds(j * CHUNK, CHUNK)]

        def g_copy(j, b):
            return pltpu.make_async_copy(table.at[sidx_of(j)], rows[b], sg[b])

        def s_copy(j, b):
            return pltpu.make_async_copy(rows[b], acc.at[didx_of(j)], ss[b])

        def c_copy(j, b):
            return pltpu.make_async_copy(ov, cacc.at[didx_of(j)], scc[b])

        for b in range(NBUF):
            rebase(b * CHUNK, CHUNK)
            g_copy(b, b).start()
        for j in range(ring, cpt):
            rebase(j * CHUNK, CHUNK)
        if tail:
            rebase(cpt * CHUNK, tail)

        for d in zd:
            d.wait()
        if with_count:
            @pl.when(c == 0)
            def _():
                for off, sz in zchunks:
                    pltpu.make_async_copy(
                        z16_h.at[pl.ds(0, sz)],
                        cacc.at[pl.ds(base + off, sz)], sz0).wait()
        plsc.subcore_barrier()

        @pl.loop(0, ring // NBUF)
        def _(i):
            for b in range(NBUF):
                j = NBUF * i + b
                bn = (b + HALF) % NBUF
                g_copy(j, b).wait()
                s_copy(j, b).start(add=True)
                if with_count:
                    @pl.when(c == 0)
                    def _():
                        c_copy(j, b).start(add=True)

                @pl.when(jnp.logical_and(j >= HALF, j + HALF < ring))
                def _():
                    s_copy(j - HALF, bn).wait()
                    if with_count:
                        @pl.when(c == 0)
                        def _():
                            c_copy(j - HALF, bn).wait()
                    rebase((j + HALF) * CHUNK, CHUNK)
                    g_copy(j + HALF, bn).start()

        for b in range(NBUF):
            s_copy(ring - NBUF + b, b).wait()
        if with_count:
            @pl.when(c == 0)
            def _():
                for b in range(NBUF):
                    c_copy(ring - NBUF + b, b).wait()

        for j in range(ring, cpt):
            pltpu.sync_copy(table.at[sidx_of(j)], rows[0])
            pltpu.sync_copy(rows[0], acc.at[didx_of(j)], add=True)
            if with_count:
                @pl.when(c == 0)
                def _():
                    pltpu.sync_copy(ov, cacc.at[didx_of(j)], add=True)
        if tail:
            ts = src_v.at[pl.ds(cpt * CHUNK, tail)]
            td = dst_v.at[pl.ds(cpt * CHUNK, tail)]
            rt = rows[0].at[pl.ds(0, tail)]
            pltpu.sync_copy(table.at[ts], rt)
            pltpu.sync_copy(rt, acc.at[td], add=True)
            if with_count:
                @pl.when(c == 0)
                def _():
                    pltpu.sync_copy(ov.at[pl.ds(0, tail)], cacc.at[td],
                                    add=True)

        plsc.subcore_barrier()

        od = []
        for off, sz in zchunks:
            od.append(pltpu.async_copy(
                acc.at[pl.ds(base + off, sz)],
                agg_o.at[c].at[pl.ds(base + off, sz)], sz0))
        if with_count:
            @pl.when(c == 0)
            def _():
                for off, sz in zchunks:
                    pltpu.async_copy(cacc.at[pl.ds(base + off, sz)],
                                     cnt_o.at[pl.ds(base + off, sz)],
                                     sz0).wait()
        for d in od:
            d.wait()

    k = pl.kernel(body, out_type=tuple(out_type), mesh=mesh,
                  scratch_types=scratch,
                  compiler_params=pltpu.CompilerParams(
                      use_tc_tiling_on_sc=False))
    zdh = jnp.zeros((128, dh), jnp.float32)
    z16 = jnp.zeros((128, 16), jnp.float32)
    o16 = jnp.ones((CHUNK, 16), jnp.float32)
    return k(table, ei_flat, zdh, z16, o16)


def _tc_layer(aggv, inv_pk, hv, wlb, wrb, bpk, relu, nc2,
              unpack_out=False, block_rows=2000):
    h_raw = hv.ndim == 2
    n2 = hv.shape[0] // 2 if h_raw else hv.shape[1]
    rp = block_rows // 2

    def body(agg_ref, inv_ref, h_ref, wlb_ref, wrb_ref, b_ref, o_ref):
        a0 = agg_ref[0] * inv_ref[...]
        a1 = agg_ref[1] * inv_ref[...]
        if h_raw:
            xr = h_ref[...].reshape(rp, 2, h_ref.shape[-1])
            xe = xr[:, 0]
            xo = xr[:, 1]
        for c2 in range(nc2):
            acc = jnp.dot(a0, wlb_ref[c2, 0],
                          preferred_element_type=jnp.float32)
            acc = acc + jnp.dot(a1, wlb_ref[c2, 1],
                                preferred_element_type=jnp.float32)
            if h_raw:
                acc = acc + jnp.concatenate(
                    [jnp.dot(xe, wrb_ref[c2],
                             preferred_element_type=jnp.float32),
                     jnp.dot(xo, wrb_ref[c2],
                             preferred_element_type=jnp.float32)], axis=-1)
            else:
                acc = acc + jnp.dot(h_ref[0], wrb_ref[c2, 0],
                                    preferred_element_type=jnp.float32)
                acc = acc + jnp.dot(h_ref[1], wrb_ref[c2, 1],
                                    preferred_element_type=jnp.float32)
            acc = acc + b_ref[c2]
            if relu:
                acc = jnp.maximum(acc, 0.0)
            if unpack_out:
                ye = acc[:, :64]
                yo = acc[:, 64:]
                o_ref[...] = jnp.stack([ye, yo], axis=1).reshape(
                    block_rows, 64)
            else:
                o_ref[c2] = acc

    if h_raw:
        din = hv.shape[1]
        h_spec = pl.BlockSpec((block_rows, din), lambda i: (i, 0))
        wr_spec = pl.BlockSpec((nc2, din, 64), lambda i: (0, 0, 0))
    else:
        h_spec = pl.BlockSpec((NC, rp, 128), lambda i: (0, i, 0))
        wr_spec = pl.BlockSpec((nc2, NC, 128, 128), lambda i: (0, 0, 0, 0))

    return pl.pallas_call(
        body,
        grid=(n2 // rp,),
        in_specs=[
            pl.BlockSpec((NC, rp, 128), lambda i: (0, i, 0)),
            pl.BlockSpec((rp, 128), lambda i: (i, 0)),
            h_spec,
            pl.BlockSpec((nc2, NC, 128, 128), lambda i: (0, 0, 0, 0)),
            wr_spec,
            pl.BlockSpec((nc2, 1, 128), lambda i: (0, 0, 0)),
        ],
        out_specs=(pl.BlockSpec((block_rows, 64), lambda i: (i, 0))
                   if unpack_out else
                   pl.BlockSpec((nc2, rp, 128), lambda i: (0, i, 0))),
        out_shape=(jax.ShapeDtypeStruct((2 * n2, 64), jnp.float32)
                   if unpack_out else
                   jax.ShapeDtypeStruct((nc2, n2, 128), jnp.float32)),
    )(aggv, inv_pk, hv, wlb, wrb, bpk)


def _bd(m):
    z = jnp.zeros_like(m)
    return jnp.concatenate(
        [jnp.concatenate([m, z], axis=1), jnp.concatenate([z, m], axis=1)],
        axis=0)


def _pack_weights(wl, wr, b, nc2, h_raw=False):
    wlb = jnp.stack([
        jnp.stack([_bd(wl[c1 * 64:(c1 + 1) * 64, c2 * 64:(c2 + 1) * 64])
                   for c1 in range(NC)])
        for c2 in range(nc2)])
    if h_raw:
        wrb = jnp.stack([wr[:, c2 * 64:(c2 + 1) * 64] for c2 in range(nc2)])
    else:
        wrb = jnp.stack([
            jnp.stack([_bd(wr[c1 * 64:(c1 + 1) * 64, c2 * 64:(c2 + 1) * 64])
                       for c1 in range(NC)])
            for c2 in range(nc2)])
    bpk = jnp.stack([
        jnp.concatenate([b[c2 * 64:(c2 + 1) * 64]] * 2)[None, :]
        for c2 in range(nc2)])
    return wlb, wrb, bpk


def kernel(x, edge_index, Wl1, Wr1, b1, Wl2, Wr2, b2, Wl3, Wr3, b3):
    n, d = x.shape
    e = edge_index.shape[1]
    n_acc = ((n + 16) + NS * 8 - 1) // (NS * 8) * (NS * 8)

    src = edge_index[0]
    dst = edge_index[1]
    align = NS * 16
    if e % align:
        pad = align - e % align
        ar = jnp.arange(pad, dtype=jnp.int32)
        src = jnp.concatenate([src, (ar * 37) % n])
        dst = jnp.concatenate([dst, n + (ar % 16)])
        e += pad
    ei_flat = jnp.concatenate([src, dst]) if e != edge_index.shape[1] \
        else edge_index.reshape(2 * e)

    dh = d // NC
    agg1, cnt = _sc_segsum(x.reshape(NC * n, dh), ei_flat, e, n_acc,
                           pair_mode=True, with_count=True, NBUF=4)
    inv = 1.0 / jnp.maximum(cnt[:n, 0], 1.0)
    inv_pk = jnp.broadcast_to(inv[:, None], (n, dh)).reshape(n // 2, 128)

    wb1 = _pack_weights(Wl1, Wr1, b1, NC, h_raw=True)
    wb2 = _pack_weights(Wl2, Wr2, b2, NC)
    wb3 = _pack_weights(Wl3, Wr3, b3, 1)

    def aggv(a):
        return a.reshape(NC, n_acc // 2, 128)

    h1 = _tc_layer(aggv(agg1), inv_pk, x, *wb1, relu=True, nc2=NC)
    (agg2,) = _sc_segsum(h1.reshape(NC * n, dh), ei_flat, e, n_acc,
                         pair_mode=False, with_count=False, NBUF=6)
    h2 = _tc_layer(aggv(agg2), inv_pk, h1, *wb2, relu=True, nc2=NC)
    (agg3,) = _sc_segsum(h2.reshape(NC * n, dh), ei_flat, e, n_acc,
                         pair_mode=False, with_count=False, NBUF=6)
    out = _tc_layer(aggv(agg3), inv_pk, h2, *wb3, relu=False, nc2=1)
    return out.reshape(n, dh)
